# baseline (speedup 1.0000x reference)
import functools

import jax
import jax.numpy as jnp
from jax import lax
from jax.experimental import pallas as pl
from jax.experimental.pallas import tpu as pltpu
from jax.experimental.pallas import tpu_sc as plsc

N_NODES = 10000
NP = 10240
N_EDGES = 320000
D = 128
NC, NS = 2, 16
NW = NC * NS
C = 128
EPT = 10240
K = EPT // C
KH = K // 2
CE = 16
XK = N_NODES // CE
XPER = -(-XK // NW)
RN = NP // NS
REP = 64

_mesh = plsc.VectorSubcoreMesh(core_axis_name="c", subcore_axis_name="s")


@functools.partial(
    pl.kernel,
    out_type=jax.ShapeDtypeStruct((NC, NP, D), jnp.float32),
    mesh=_mesh,
    scratch_types=[
        pltpu.VMEM((KH, C), jnp.int32),
        pltpu.VMEM((KH, C), jnp.int32),
        pltpu.VMEM((C, D), jnp.float32),
        pltpu.VMEM((C, D), jnp.float32),
        pltpu.VMEM_SHARED((NP, D), jnp.float32),
        pltpu.SemaphoreType.DMA,
        pltpu.SemaphoreType.DMA,
        pltpu.SemaphoreType.DMA,
        pltpu.SemaphoreType.DMA,
    ],
)
def _sc_scatter(table_hbm, init_hbm, z_hbm, rows_hbm, cols_hbm, out_hbm,
                row_v, col_v, msg0_v, msg1_v, acc, sem0, sem1, sems0, sems1):
    c = lax.axis_index("c")
    s = lax.axis_index("s")
    wid = s * NC + c

    @pl.when(c == 0)
    def _():
        pltpu.sync_copy(init_hbm.at[pl.ds(s * RN, RN)], acc.at[pl.ds(s * RN, RN)])

    @pl.when(c == 1)
    def _():
        pltpu.sync_copy(z_hbm.at[pl.ds(s * RN, RN)], acc.at[pl.ds(s * RN, RN)])

    plsc.subcore_barrier()

    for hh in range(2):
        pltpu.sync_copy(rows_hbm.at[wid, hh], row_v)
        pltpu.sync_copy(cols_hbm.at[wid, hh], col_v)
        pltpu.async_copy(table_hbm.at[row_v.at[0]], msg0_v, sem0)
        pltpu.async_copy(table_hbm.at[row_v.at[1]], msg1_v, sem1)

        @pl.loop(0, KH, step=2)
        def _(j):
            pltpu.make_async_copy(table_hbm.at[row_v.at[j]], msg0_v, sem0).wait()
            pltpu.async_copy(msg0_v, acc.at[col_v.at[j]], sems0, add=True)
            pltpu.make_async_copy(table_hbm.at[row_v.at[j + 1]], msg1_v, sem1).wait()
            pltpu.async_copy(msg1_v, acc.at[col_v.at[j + 1]], sems1, add=True)
            pltpu.make_async_copy(msg0_v, acc.at[col_v.at[j]], sems0).wait()

            @pl.when(j + 2 < KH)
            def _():
                pltpu.async_copy(table_hbm.at[row_v.at[j + 2]], msg0_v, sem0)

            pltpu.make_async_copy(msg1_v, acc.at[col_v.at[j + 1]], sems1).wait()

            @pl.when(j + 3 < KH)
            def _():
                pltpu.async_copy(table_hbm.at[row_v.at[j + 3]], msg1_v, sem1)

    plsc.subcore_barrier()
    pltpu.sync_copy(acc.at[pl.ds(s * RN, RN)], out_hbm.at[c, pl.ds(s * RN, RN)])


@functools.partial(
    pl.kernel,
    out_type=jax.ShapeDtypeStruct((NP, D), jnp.float32),
    mesh=_mesh,
    scratch_types=[
        pltpu.VMEM((1, CE), jnp.int32),
        pltpu.VMEM((CE, D), jnp.float32),
        pltpu.SemaphoreType.DMA,
    ],
)
def _sc_embed(cx_hbm, xtab_hbm, h0_out, xidx_v, xrow_v, sem):
    c = lax.axis_index("c")
    s = lax.axis_index("s")
    wid = s * NC + c
    lo = wid * XPER
    hi = jnp.minimum(lo + XPER, XK)

    @pl.loop(lo, hi)
    def _(j):
        pltpu.sync_copy(cx_hbm.at[j], xidx_v)
        pltpu.async_copy(xtab_hbm.at[xidx_v.at[0]], xrow_v, sem).wait()
        pltpu.sync_copy(xrow_v, h0_out.at[pl.ds(j * CE, CE)])


def _mlp_layer(S, cnt, ec, w1l, b1l, w2l, b2l, relu_out):
    R = 1024
    grid = (NP // R,)

    def body(s_ref, c_ref, ec_ref, w1_ref, b1_ref, w2_ref, b2_ref, o_ref):
        agg = s_ref[0] + s_ref[1]
        cb = c_ref[0] + c_ref[1]
        agg = agg + jnp.dot(cb, ec_ref[...], preferred_element_type=jnp.float32)
        hmid = jnp.dot(agg, w1_ref[...], preferred_element_type=jnp.float32)
        hmid = jnp.maximum(hmid + b1_ref[...], 0.0)
        out = jnp.dot(hmid, w2_ref[...], preferred_element_type=jnp.float32)
        out = out + b2_ref[...]
        if relu_out:
            out = jnp.maximum(out, 0.0)
        o_ref[...] = out

    return pl.pallas_call(
        body,
        grid=grid,
        in_specs=[
            pl.BlockSpec((NC, R, D), lambda i: (0, i, 0)),
            pl.BlockSpec((NC, R, D), lambda i: (0, i, 0)),
            pl.BlockSpec((D, D), lambda i: (0, 0)),
            pl.BlockSpec((D, 2 * D), lambda i: (0, 0)),
            pl.BlockSpec((1, 2 * D), lambda i: (0, 0)),
            pl.BlockSpec((2 * D, D), lambda i: (0, 0)),
            pl.BlockSpec((1, D), lambda i: (0, 0)),
        ],
        out_specs=pl.BlockSpec((R, D), lambda i: (i, 0)),
        out_shape=jax.ShapeDtypeStruct((NP, D), jnp.float32),
    )(S, cnt, ec, w1l, b1l.reshape(1, 2 * D), w2l, b2l.reshape(1, D))


def kernel(x, edge_index, edge_attr, xemb1, xemb2, eemb1, eemb2, w1, b1, w2, b2):
    npad = EPT - N_EDGES // NW
    ar = jnp.arange(npad, dtype=jnp.int32)
    drow = jnp.broadcast_to(ar % N_NODES, (NW, npad))
    dcol = jnp.broadcast_to(N_NODES + ar, (NW, npad))
    dcomb = jnp.broadcast_to((ar % REP) * 24, (NW, npad))

    def _pad(a, d):
        return jnp.concatenate([a.reshape(NW, N_EDGES // NW), d],
                               axis=1).reshape(NW, 2, KH, C)

    row = _pad(edge_index[0], drow)
    col = _pad(edge_index[1], dcol)
    rep = jnp.arange(N_EDGES, dtype=jnp.int32) % REP
    comb = _pad(edge_attr[:, 0] * 3 + edge_attr[:, 1] + 24 * rep, dcomb)
    cx = (x[:, 0] * 3 + x[:, 1]).reshape(XK, 1, CE)

    xtab = (xemb1[:, None, :] + xemb2[None, :, :]).reshape(-1, D)
    onehot = jnp.tile(jnp.eye(24, D, dtype=jnp.float32), (REP, 1))
    selfloop = jnp.zeros((NP, D), jnp.float32).at[:, 12].set(1.0)
    z128 = jnp.zeros((NP, D), jnp.float32)
    ec = (eemb1[:, :, None, :] + eemb2[:, None, :, :]).reshape(5, 18, D)
    ecp = jnp.zeros((5, D, D), jnp.float32).at[:, :18].set(ec)

    h = _sc_embed(cx, xtab)
    cnt = _sc_scatter(onehot, selfloop, z128, comb, col)
    nl = w1.shape[0]
    for l in range(nl):
        S = _sc_scatter(h, h, z128, row, col)
        h = _mlp_layer(S, cnt, ecp[l], w1[l], b1[l], w2[l], b2[l], l < nl - 1)
    return h[:N_NODES]


# speedup vs baseline: 1.1076x; 1.1076x over previous
import functools

import jax
import jax.numpy as jnp
from jax import lax
from jax.experimental import pallas as pl
from jax.experimental.pallas import tpu as pltpu
from jax.experimental.pallas import tpu_sc as plsc

N_NODES = 10000
NP = 10240
N_EDGES = 320000
D = 128
NC, NS = 2, 16
NW = NC * NS
C = 128
EPT = 10240
K = EPT // C
KH = K // 2
CE = 16
XK = N_NODES // CE
XPER = -(-XK // NW)
RN = NP // NS
REP = 256

_mesh = plsc.VectorSubcoreMesh(core_axis_name="c", subcore_axis_name="s")


@functools.partial(
    pl.kernel,
    out_type=jax.ShapeDtypeStruct((NC, NP, D), jnp.float32),
    mesh=_mesh,
    scratch_types=[
        pltpu.VMEM((KH, C), jnp.int32),
        pltpu.VMEM((KH, C), jnp.int32),
        pltpu.VMEM((C, D), jnp.float32),
        pltpu.VMEM((C, D), jnp.float32),
        pltpu.VMEM_SHARED((NP, D), jnp.float32),
        pltpu.SemaphoreType.DMA,
        pltpu.SemaphoreType.DMA,
    ],
)
def _sc_scatter(table_hbm, init_hbm, z_hbm, rows_hbm, cols_hbm, out_hbm,
                row_v, col_v, msg0_v, msg1_v, acc, sem0, sem1):
    c = lax.axis_index("c")
    s = lax.axis_index("s")
    wid = s * NC + c

    @pl.when(c == 0)
    def _():
        pltpu.sync_copy(init_hbm.at[pl.ds(s * RN, RN)], acc.at[pl.ds(s * RN, RN)])

    @pl.when(c == 1)
    def _():
        pltpu.sync_copy(z_hbm.at[pl.ds(s * RN, RN)], acc.at[pl.ds(s * RN, RN)])

    plsc.subcore_barrier()

    for hh in range(2):
        pltpu.sync_copy(rows_hbm.at[wid, hh], row_v)
        pltpu.sync_copy(cols_hbm.at[wid, hh], col_v)
        pltpu.async_copy(table_hbm.at[row_v.at[0]], msg0_v, sem0)

        @pl.loop(0, KH, step=2)
        def _(j):
            pltpu.make_async_copy(table_hbm.at[row_v.at[j]], msg0_v, sem0).wait()
            pltpu.async_copy(table_hbm.at[row_v.at[j + 1]], msg1_v, sem1)
            pltpu.sync_copy(msg0_v, acc.at[col_v.at[j]], add=True)
            pltpu.make_async_copy(table_hbm.at[row_v.at[j + 1]], msg1_v, sem1).wait()

            @pl.when(j + 2 < KH)
            def _():
                pltpu.async_copy(table_hbm.at[row_v.at[j + 2]], msg0_v, sem0)

            pltpu.sync_copy(msg1_v, acc.at[col_v.at[j + 1]], add=True)

    plsc.subcore_barrier()
    pltpu.sync_copy(acc.at[pl.ds(s * RN, RN)], out_hbm.at[c, pl.ds(s * RN, RN)])


@functools.partial(
    pl.kernel,
    out_type=jax.ShapeDtypeStruct((NP, D), jnp.float32),
    mesh=_mesh,
    scratch_types=[
        pltpu.VMEM((1, CE), jnp.int32),
        pltpu.VMEM((CE, D), jnp.float32),
        pltpu.SemaphoreType.DMA,
    ],
)
def _sc_embed(cx_hbm, xtab_hbm, h0_out, xidx_v, xrow_v, sem):
    c = lax.axis_index("c")
    s = lax.axis_index("s")
    wid = s * NC + c
    lo = wid * XPER
    hi = jnp.minimum(lo + XPER, XK)

    @pl.loop(lo, hi)
    def _(j):
        pltpu.sync_copy(cx_hbm.at[j], xidx_v)
        pltpu.async_copy(xtab_hbm.at[xidx_v.at[0]], xrow_v, sem).wait()
        pltpu.sync_copy(xrow_v, h0_out.at[pl.ds(j * CE, CE)])


def _mlp_layer(S, cnt, ec, w1l, b1l, w2l, b2l, relu_out, first):
    R = 1024
    grid = (NP // R,)

    def body(s_ref, c_ref, ec_ref, w1_ref, b1_ref, w2_ref, b2_ref, *o_refs):
        agg = s_ref[0] + s_ref[1]
        if first:
            cb = c_ref[0] + c_ref[1]
            o_refs[1][...] = cb
        else:
            cb = c_ref[...]
        agg = agg + jnp.dot(cb, ec_ref[...], preferred_element_type=jnp.float32)
        hmid = jnp.dot(agg, w1_ref[...], preferred_element_type=jnp.float32)
        hmid = jnp.maximum(hmid + b1_ref[...], 0.0)
        out = jnp.dot(hmid, w2_ref[...], preferred_element_type=jnp.float32)
        out = out + b2_ref[...]
        if relu_out:
            out = jnp.maximum(out, 0.0)
        o_refs[0][...] = out

    cnt_spec = (pl.BlockSpec((NC, R, D), lambda i: (0, i, 0)) if first
                else pl.BlockSpec((R, D), lambda i: (i, 0)))
    out_specs = pl.BlockSpec((R, D), lambda i: (i, 0))
    out_shape = jax.ShapeDtypeStruct((NP, D), jnp.float32)
    if first:
        out_specs = [out_specs, pl.BlockSpec((R, D), lambda i: (i, 0))]
        out_shape = [out_shape, jax.ShapeDtypeStruct((NP, D), jnp.float32)]
    return pl.pallas_call(
        body,
        grid=grid,
        in_specs=[
            pl.BlockSpec((NC, R, D), lambda i: (0, i, 0)),
            cnt_spec,
            pl.BlockSpec((D, D), lambda i: (0, 0)),
            pl.BlockSpec((D, 2 * D), lambda i: (0, 0)),
            pl.BlockSpec((1, 2 * D), lambda i: (0, 0)),
            pl.BlockSpec((2 * D, D), lambda i: (0, 0)),
            pl.BlockSpec((1, D), lambda i: (0, 0)),
        ],
        out_specs=out_specs,
        out_shape=out_shape,
    )(S, cnt, ec, w1l, b1l.reshape(1, 2 * D), w2l, b2l.reshape(1, D))


def kernel(x, edge_index, edge_attr, xemb1, xemb2, eemb1, eemb2, w1, b1, w2, b2):
    npad = EPT - N_EDGES // NW
    ar = jnp.arange(npad, dtype=jnp.int32)
    drow = jnp.broadcast_to(ar % N_NODES, (NW, npad))
    dcol = jnp.broadcast_to(N_NODES + ar, (NW, npad))
    dcomb = jnp.broadcast_to((ar % REP) * 24, (NW, npad))

    def _pad(a, d):
        return jnp.concatenate([a.reshape(NW, N_EDGES // NW), d],
                               axis=1).reshape(NW, 2, KH, C)

    row = _pad(edge_index[0], drow)
    col = _pad(edge_index[1], dcol)
    rep = jnp.arange(N_EDGES, dtype=jnp.int32) % REP
    comb = _pad(edge_attr[:, 0] * 3 + edge_attr[:, 1] + 24 * rep, dcomb)
    cx = (x[:, 0] * 3 + x[:, 1]).reshape(XK, 1, CE)

    xtab = (xemb1[:, None, :] + xemb2[None, :, :]).reshape(-1, D)
    onehot = jnp.tile(jnp.eye(24, D, dtype=jnp.float32), (REP, 1))
    selfloop = jnp.zeros((NP, D), jnp.float32).at[:, 12].set(1.0)
    z128 = jnp.zeros((NP, D), jnp.float32)
    ec = (eemb1[:, :, None, :] + eemb2[:, None, :, :]).reshape(5, 18, D)
    ecp = jnp.zeros((5, D, D), jnp.float32).at[:, :18].set(ec)

    h = _sc_embed(cx, xtab)
    cnt = _sc_scatter(onehot, selfloop, z128, comb, col)
    nl = w1.shape[0]
    for l in range(nl):
        S = _sc_scatter(h, h, z128, row, col)
        if l == 0:
            h, cnt = _mlp_layer(S, cnt, ecp[l], w1[l], b1[l], w2[l], b2[l],
                                True, True)
        else:
            h = _mlp_layer(S, cnt, ecp[l], w1[l], b1[l], w2[l], b2[l],
                           l < nl - 1, False)
    return h[:N_NODES]


# speedup vs baseline: 1.1129x; 1.0048x over previous
import functools

import jax
import jax.numpy as jnp
from jax import lax
from jax.experimental import pallas as pl
from jax.experimental.pallas import tpu as pltpu
from jax.experimental.pallas import tpu_sc as plsc

N_NODES = 10000
NP = 10240
N_EDGES = 320000
D = 128
NC, NS = 2, 16
NW = NC * NS
C = 128
EPT = 10240
K = EPT // C
KH = K // 2
CE = 80
XK = N_NODES // CE
XPER = -(-XK // NW)
RN = NP // NS
REP = 256

_mesh = plsc.VectorSubcoreMesh(core_axis_name="c", subcore_axis_name="s")


@functools.partial(
    pl.kernel,
    out_type=jax.ShapeDtypeStruct((NC, NP, D), jnp.float32),
    mesh=_mesh,
    scratch_types=[
        pltpu.VMEM((KH, C), jnp.int32),
        pltpu.VMEM((KH, C), jnp.int32),
        pltpu.VMEM((C, D), jnp.float32),
        pltpu.VMEM((C, D), jnp.float32),
        pltpu.VMEM_SHARED((NP, D), jnp.float32),
        pltpu.SemaphoreType.DMA,
        pltpu.SemaphoreType.DMA,
    ],
)
def _sc_scatter(table_hbm, init_hbm, z_hbm, rows_hbm, cols_hbm, out_hbm,
                row_v, col_v, msg0_v, msg1_v, acc, sem0, sem1):
    c = lax.axis_index("c")
    s = lax.axis_index("s")
    wid = s * NC + c

    @pl.when(c == 0)
    def _():
        pltpu.sync_copy(init_hbm.at[pl.ds(s * RN, RN)], acc.at[pl.ds(s * RN, RN)])

    @pl.when(c == 1)
    def _():
        pltpu.sync_copy(z_hbm.at[pl.ds(s * RN, RN)], acc.at[pl.ds(s * RN, RN)])

    plsc.subcore_barrier()

    for hh in range(2):
        pltpu.sync_copy(rows_hbm.at[wid, hh], row_v)
        pltpu.sync_copy(cols_hbm.at[wid, hh], col_v)
        pltpu.async_copy(table_hbm.at[row_v.at[0]], msg0_v, sem0)

        @pl.loop(0, KH, step=2)
        def _(j):
            pltpu.make_async_copy(table_hbm.at[row_v.at[j]], msg0_v, sem0).wait()
            pltpu.async_copy(table_hbm.at[row_v.at[j + 1]], msg1_v, sem1)
            pltpu.sync_copy(msg0_v, acc.at[col_v.at[j]], add=True)
            pltpu.make_async_copy(table_hbm.at[row_v.at[j + 1]], msg1_v, sem1).wait()

            @pl.when(j + 2 < KH)
            def _():
                pltpu.async_copy(table_hbm.at[row_v.at[j + 2]], msg0_v, sem0)

            pltpu.sync_copy(msg1_v, acc.at[col_v.at[j + 1]], add=True)

    plsc.subcore_barrier()
    pltpu.sync_copy(acc.at[pl.ds(s * RN, RN)], out_hbm.at[c, pl.ds(s * RN, RN)])


@functools.partial(
    pl.kernel,
    out_type=jax.ShapeDtypeStruct((NP, D), jnp.float32),
    mesh=_mesh,
    scratch_types=[
        pltpu.VMEM((1, CE), jnp.int32),
        pltpu.VMEM((CE, D), jnp.float32),
        pltpu.SemaphoreType.DMA,
    ],
)
def _sc_embed(cx_hbm, xtab_hbm, h0_out, xidx_v, xrow_v, sem):
    c = lax.axis_index("c")
    s = lax.axis_index("s")
    wid = s * NC + c
    lo = wid * XPER
    hi = jnp.minimum(lo + XPER, XK)

    @pl.loop(lo, hi)
    def _(j):
        pltpu.sync_copy(cx_hbm.at[j], xidx_v)
        pltpu.async_copy(xtab_hbm.at[xidx_v.at[0]], xrow_v, sem).wait()
        pltpu.sync_copy(xrow_v, h0_out.at[pl.ds(j * CE, CE)])


def _mlp_layer(S, cnt, ec, w1l, b1l, w2l, b2l, relu_out, first):
    R = 1024
    grid = (NP // R,)

    def body(s_ref, c_ref, ec_ref, w1_ref, b1_ref, w2_ref, b2_ref, *o_refs):
        agg = s_ref[0] + s_ref[1]
        if first:
            cb = c_ref[0] + c_ref[1]
            o_refs[1][...] = cb
        else:
            cb = c_ref[...]
        agg = agg + jnp.dot(cb, ec_ref[...], preferred_element_type=jnp.float32)
        hmid = jnp.dot(agg, w1_ref[...], preferred_element_type=jnp.float32)
        hmid = jnp.maximum(hmid + b1_ref[...], 0.0)
        out = jnp.dot(hmid, w2_ref[...], preferred_element_type=jnp.float32)
        out = out + b2_ref[...]
        if relu_out:
            out = jnp.maximum(out, 0.0)
        o_refs[0][...] = out

    cnt_spec = (pl.BlockSpec((NC, R, D), lambda i: (0, i, 0)) if first
                else pl.BlockSpec((R, D), lambda i: (i, 0)))
    out_specs = pl.BlockSpec((R, D), lambda i: (i, 0))
    out_shape = jax.ShapeDtypeStruct((NP, D), jnp.float32)
    if first:
        out_specs = [out_specs, pl.BlockSpec((R, D), lambda i: (i, 0))]
        out_shape = [out_shape, jax.ShapeDtypeStruct((NP, D), jnp.float32)]
    return pl.pallas_call(
        body,
        grid=grid,
        in_specs=[
            pl.BlockSpec((NC, R, D), lambda i: (0, i, 0)),
            cnt_spec,
            pl.BlockSpec((D, D), lambda i: (0, 0)),
            pl.BlockSpec((D, 2 * D), lambda i: (0, 0)),
            pl.BlockSpec((1, 2 * D), lambda i: (0, 0)),
            pl.BlockSpec((2 * D, D), lambda i: (0, 0)),
            pl.BlockSpec((1, D), lambda i: (0, 0)),
        ],
        out_specs=out_specs,
        out_shape=out_shape,
    )(S, cnt, ec, w1l, b1l.reshape(1, 2 * D), w2l, b2l.reshape(1, D))


def kernel(x, edge_index, edge_attr, xemb1, xemb2, eemb1, eemb2, w1, b1, w2, b2):
    npad = EPT - N_EDGES // NW
    ar = jnp.arange(npad, dtype=jnp.int32)
    drow = jnp.broadcast_to(ar % N_NODES, (NW, npad))
    dcol = jnp.broadcast_to(N_NODES + ar, (NW, npad))
    dcomb = jnp.broadcast_to((ar % REP) * 24, (NW, npad))

    def _pad(a, d):
        return jnp.concatenate([a.reshape(NW, N_EDGES // NW), d],
                               axis=1).reshape(NW, 2, KH, C)

    row = _pad(edge_index[0], drow)
    col = _pad(edge_index[1], dcol)
    rep = jnp.arange(N_EDGES, dtype=jnp.int32) % REP
    comb = _pad(edge_attr[:, 0] * 3 + edge_attr[:, 1] + 24 * rep, dcomb)
    cx = (x[:, 0] * 3 + x[:, 1]).reshape(XK, 1, CE)

    xtab = (xemb1[:, None, :] + xemb2[None, :, :]).reshape(-1, D)
    onehot = jnp.tile(jnp.eye(24, D, dtype=jnp.float32), (REP, 1))
    selfloop = jnp.zeros((NP, D), jnp.float32).at[:, 12].set(1.0)
    z128 = jnp.zeros((NP, D), jnp.float32)
    ec = (eemb1[:, :, None, :] + eemb2[:, None, :, :]).reshape(5, 18, D)
    ecp = jnp.zeros((5, D, D), jnp.float32).at[:, :18].set(ec)

    h = _sc_embed(cx, xtab)
    cnt = _sc_scatter(onehot, selfloop, z128, comb, col)
    nl = w1.shape[0]
    for l in range(nl):
        S = _sc_scatter(h, h, z128, row, col)
        if l == 0:
            h, cnt = _mlp_layer(S, cnt, ecp[l], w1[l], b1[l], w2[l], b2[l],
                                True, True)
        else:
            h = _mlp_layer(S, cnt, ecp[l], w1[l], b1[l], w2[l], b2[l],
                           l < nl - 1, False)
    return h[:N_NODES]
